# baseline (speedup 1.0000x reference)
import functools

import jax
import jax.numpy as jnp
from jax import lax
from jax.experimental import pallas as pl
from jax.experimental.pallas import tpu as pltpu
from jax.experimental.pallas import tpu_sc as plsc


def _tc_body(idx_ref, d_ref, w1_ref, b1_ref, w2t_ref, b2_ref, out_ref):
    B3, S3, K = idx_ref.shape
    R = B3 * S3
    HI = lax.Precision.HIGHEST
    idx = idx_ref[...].reshape(R, K)
    d = d_ref[...].reshape(R, K)
    idxf = idx.astype(jnp.float32)
    kk = lax.broadcasted_iota(jnp.int32, (K, K * K), 0)
    ll = lax.broadcasted_iota(jnp.int32, (K, K * K), 1)
    arep = (ll // K == kk).astype(jnp.float32)
    brep = (ll % K == kk).astype(jnp.float32)
    a2 = jnp.dot(idxf, arep, precision=HI)
    b2 = jnp.dot(idxf, brep, precision=HI)
    eq2 = (a2 == b2).astype(jnp.float32)
    l1 = lax.broadcasted_iota(jnp.int32, (1, K * K), 1)
    ltm = (l1 % K < l1 // K).astype(jnp.float32)
    sred = arep.T
    seen = jnp.dot(eq2 * ltm, sred, precision=HI)
    is_new = ((idx != 0) & (seen == 0.0)).astype(jnp.float32)
    r0 = lax.broadcasted_iota(jnp.int32, (K, K), 0)
    r1 = lax.broadcasted_iota(jnp.int32, (K, K), 1)
    tri = (r0 <= r1).astype(jnp.float32)
    counts = jnp.dot(is_new, tri, precision=HI)
    feat = jnp.concatenate([d, counts], axis=-1)
    h = jnp.tanh(jnp.dot(feat, w1_ref[...], precision=HI) + b1_ref[...])
    logit = jnp.sum(h * w2t_ref[...], axis=-1, keepdims=True) + b2_ref[...]
    tempe = jax.nn.sigmoid(logit)
    x = -d * tempe
    x = x - jnp.max(x, axis=-1, keepdims=True)
    e = jnp.exp(x)
    p = e / jnp.sum(e, axis=-1, keepdims=True)
    p2 = jnp.dot(p, brep, precision=HI)
    comb = jnp.dot(eq2 * p2, sred, precision=HI)
    out_ref[...] = comb


def _tc_combine(idx3, d3, W1, b1, W2, b2):
    B3, S3, K = idx3.shape
    return pl.pallas_call(
        _tc_body,
        out_shape=jax.ShapeDtypeStruct((B3 * S3, K), jnp.float32),
    )(idx3, d3, W1, b1.reshape(1, -1), W2.reshape(1, -1), b2.reshape(1, 1))


@functools.cache
def _make_sc_scatter(B, S, K, V):
    NC, NS = 2, 16
    NW = NC * NS
    assert B == NW and K % 16 == 0
    CW = 6144
    NCHUNK = V // CW
    TW = V - NCHUNK * CW
    mesh = plsc.VectorSubcoreMesh(core_axis_name="c", subcore_axis_name="s")

    def _scatter_halves(buf, idx_v, val_v, base, width, vals_are_zero):
        for s in range(S):
            srow = jnp.full((16,), s, jnp.int32)
            for h in range(K // 16):
                iv = idx_v[s, pl.ds(h * 16, 16)]
                m = (iv >= base) & (iv < base + width)
                loc = jnp.where(m, iv - base, 0)
                if vals_are_zero:
                    vv = jnp.zeros((16,), jnp.float32)
                else:
                    vv = val_v[s, pl.ds(h * 16, 16)]
                plsc.store_scatter(buf, [srow, loc], vv, mask=m)

    @functools.partial(
        pl.kernel,
        mesh=mesh,
        out_type=jax.ShapeDtypeStruct((B, S, V), jnp.float32),
        compiler_params=pltpu.CompilerParams(needs_layout_passes=False),
        scratch_types=[
            pltpu.VMEM((S, CW), jnp.float32),
            pltpu.VMEM((S, CW), jnp.float32),
            pltpu.VMEM((S, TW), jnp.float32),
            pltpu.VMEM((S, K), jnp.int32),
            pltpu.VMEM((S, K), jnp.float32),
            pltpu.SemaphoreType.DMA,
            pltpu.SemaphoreType.DMA,
            pltpu.SemaphoreType.DMA,
        ],
    )
    def sc_scatter(zeros_hbm, idx_hbm, val_hbm, out_hbm,
                   buf_a, buf_b, tailbuf, idx_v, val_v, sem_a, sem_b, sem_p):
        b = lax.axis_index("s") * NC + lax.axis_index("c")
        bufs = (buf_a, buf_b)
        sems = (sem_a, sem_b)
        pre = [
            pltpu.async_copy(zeros_hbm.at[:, pl.ds(0, CW)], buf_a, sem_p),
            pltpu.async_copy(zeros_hbm.at[:, pl.ds(0, CW)], buf_b, sem_p),
            pltpu.async_copy(zeros_hbm.at[:, pl.ds(CW, TW)], tailbuf, sem_p),
            pltpu.async_copy(idx_hbm.at[pl.ds(b * S, S)], idx_v, sem_p),
            pltpu.async_copy(val_hbm.at[pl.ds(b * S, S)], val_v, sem_p),
        ]
        for cp in pre:
            cp.wait()
        copies = [None, None]
        for c in range(NCHUNK):
            buf = bufs[c % 2]
            if c >= 2:
                copies[c % 2].wait()
                _scatter_halves(buf, idx_v, val_v, (c - 2) * CW, CW, True)
            _scatter_halves(buf, idx_v, val_v, c * CW, CW, False)
            copies[c % 2] = pltpu.async_copy(
                buf, out_hbm.at[b, :, pl.ds(c * CW, CW)], sems[c % 2])
        base = NCHUNK * CW
        _scatter_halves(tailbuf, idx_v, val_v, base, TW, False)
        tail_cp = pltpu.async_copy(
            tailbuf, out_hbm.at[b, :, pl.ds(base, TW)], sem_p)
        copies[0].wait()
        copies[1].wait()
        tail_cp.wait()

    return sc_scatter


def kernel(tgt_index, knn_dists, nmt_prob, W1, b1, W2, b2):
    B, S, K = knn_dists.shape
    V = nmt_prob.shape[-1]
    R = B * S
    idx3 = tgt_index.astype(jnp.int32)
    idx = idx3.reshape(R, K)
    d3 = knn_dists.astype(jnp.float32)
    vals = _tc_combine(idx3, d3, W1, b1, W2, b2)
    CW, TW = 6144, V - (V // 6144) * 6144
    zeros_src = jnp.zeros((S, CW + TW), jnp.float32)
    return _make_sc_scatter(B, S, K, V)(zeros_src, idx, vals)


# speedup vs baseline: 1.0879x; 1.0879x over previous
import functools

import jax
import jax.numpy as jnp
from jax import lax
from jax.experimental import pallas as pl
from jax.experimental.pallas import tpu as pltpu
from jax.experimental.pallas import tpu_sc as plsc


def _tc_body(idx_ref, d_ref, w1_ref, b1_ref, w2t_ref, b2_ref, out_ref):
    B3, S3, K = idx_ref.shape
    R = B3 * S3
    idx = idx_ref[...].reshape(R, K)
    d = d_ref[...].reshape(R, K)
    eq = idx[:, :, None] == idx[:, None, :]
    r0 = lax.broadcasted_iota(jnp.int32, (K, K), 0)
    r1 = lax.broadcasted_iota(jnp.int32, (K, K), 1)
    seen = jnp.sum(jnp.where(eq & (r1 < r0)[None], 1, 0), axis=-1) > 0
    is_new = ((idx != 0) & ~seen).astype(jnp.float32)
    tri = (r0 <= r1).astype(jnp.float32)
    counts = jnp.dot(is_new, tri, precision=lax.Precision.HIGHEST)
    feat = jnp.concatenate([d, counts], axis=-1)
    h = jnp.tanh(
        jnp.dot(feat, w1_ref[...], precision=lax.Precision.HIGHEST)
        + b1_ref[...]
    )
    logit = jnp.sum(h * w2t_ref[...], axis=-1, keepdims=True) + b2_ref[...]
    tempe = jax.nn.sigmoid(logit)
    x = -d * tempe
    x = x - jnp.max(x, axis=-1, keepdims=True)
    e = jnp.exp(x)
    p = e / jnp.sum(e, axis=-1, keepdims=True)
    comb = jnp.sum(eq.astype(jnp.float32) * p[:, None, :], axis=-1)
    out_ref[...] = comb.reshape(B3, S3, K)


def _tc_combine(idx3, d3, W1, b1, W2, b2):
    B3, S3, K = idx3.shape
    return pl.pallas_call(
        _tc_body,
        out_shape=jax.ShapeDtypeStruct((B3, S3, K), jnp.float32),
    )(idx3, d3, W1, b1.reshape(1, -1), W2.reshape(1, -1), b2.reshape(1, 1))


@functools.cache
def _make_sc_scatter(B, S, K, V):
    NC, NS = 2, 16
    NW = NC * NS
    assert B == NW and K % 16 == 0
    CW = 6144
    NCHUNK = V // CW
    assert NCHUNK % 2 == 0
    TW = V - NCHUNK * CW
    mesh = plsc.VectorSubcoreMesh(core_axis_name="c", subcore_axis_name="s")

    def _scatter_halves(buf, idx_v, val_v, base, width, vals_are_zero):
        for s in range(S):
            srow = jnp.full((16,), s, jnp.int32)
            for h in range(K // 16):
                iv = idx_v[s, pl.ds(h * 16, 16)]
                m = (iv >= base) & (iv < base + width)
                loc = jnp.where(m, iv - base, 0)
                if vals_are_zero:
                    vv = jnp.zeros((16,), jnp.float32)
                else:
                    vv = val_v[s, pl.ds(h * 16, 16)]
                plsc.store_scatter(buf, [srow, loc], vv, mask=m)

    @functools.partial(
        pl.kernel,
        mesh=mesh,
        out_type=jax.ShapeDtypeStruct((B, S, V), jnp.float32),
        compiler_params=pltpu.CompilerParams(needs_layout_passes=False),
        scratch_types=[
            pltpu.VMEM((S, CW), jnp.float32),
            pltpu.VMEM((S, CW), jnp.float32),
            pltpu.VMEM((S, TW), jnp.float32),
            pltpu.VMEM((S, K), jnp.int32),
            pltpu.VMEM((S, K), jnp.float32),
            pltpu.SemaphoreType.DMA,
            pltpu.SemaphoreType.DMA,
            pltpu.SemaphoreType.DMA,
        ],
    )
    def sc_scatter(zeros_hbm, idx_hbm, val_hbm, out_hbm,
                   buf_a, buf_b, tailbuf, idx_v, val_v, sem_a, sem_b, sem_p):
        b = lax.axis_index("s") * NC + lax.axis_index("c")
        pre = [
            pltpu.async_copy(zeros_hbm.at[:, pl.ds(0, CW)], buf_a, sem_p),
            pltpu.async_copy(zeros_hbm.at[:, pl.ds(0, CW)], buf_b, sem_p),
            pltpu.async_copy(zeros_hbm.at[:, pl.ds(CW, TW)], tailbuf, sem_p),
            pltpu.async_copy(idx_hbm.at[b], idx_v, sem_p),
            pltpu.async_copy(val_hbm.at[b], val_v, sem_p),
        ]
        for cp in pre:
            cp.wait()

        def _fire(buf, base, sem):
            return pltpu.async_copy(
                buf, out_hbm.at[b, :, pl.ds(base, CW)], sem)

        _scatter_halves(buf_a, idx_v, val_v, 0, CW, False)
        _fire(buf_a, 0, sem_a)
        _scatter_halves(buf_b, idx_v, val_v, CW, CW, False)
        _fire(buf_b, CW, sem_b)

        @pl.loop(1, NCHUNK // 2)
        def _chunk_pair(i):
            for buf, sem, par in ((buf_a, sem_a, 0), (buf_b, sem_b, 1)):
                base = (2 * i + par) * CW
                pltpu.make_async_copy(
                    buf, out_hbm.at[b, :, pl.ds(base - 2 * CW, CW)], sem
                ).wait()
                _scatter_halves(buf, idx_v, val_v, base - 2 * CW, CW, True)
                _scatter_halves(buf, idx_v, val_v, base, CW, False)
                _fire(buf, base, sem)

        base = NCHUNK * CW
        _scatter_halves(tailbuf, idx_v, val_v, base, TW, False)
        tail_cp = pltpu.async_copy(
            tailbuf, out_hbm.at[b, :, pl.ds(base, TW)], sem_p)
        pltpu.make_async_copy(
            buf_a, out_hbm.at[b, :, pl.ds(0, CW)], sem_a).wait()
        pltpu.make_async_copy(
            buf_b, out_hbm.at[b, :, pl.ds(0, CW)], sem_b).wait()
        tail_cp.wait()

    return sc_scatter


def kernel(tgt_index, knn_dists, nmt_prob, W1, b1, W2, b2):
    B, S, K = knn_dists.shape
    V = nmt_prob.shape[-1]
    idx3 = tgt_index.astype(jnp.int32)
    d3 = knn_dists.astype(jnp.float32)
    vals3 = _tc_combine(idx3, d3, W1, b1, W2, b2)
    CW = 6144
    TW = V - (V // CW) * CW
    zeros_src = jnp.zeros((S, CW + TW), jnp.float32)
    return _make_sc_scatter(B, S, K, V)(zeros_src, idx3, vals3)


# speedup vs baseline: 1.0886x; 1.0006x over previous
import functools

import jax
import jax.numpy as jnp
from jax import lax
from jax.experimental import pallas as pl
from jax.experimental.pallas import tpu as pltpu
from jax.experimental.pallas import tpu_sc as plsc


def _tc_body(idx_ref, d_ref, w1_ref, b1_ref, w2t_ref, b2_ref, out_ref):
    R, K = idx_ref.shape
    HI = lax.Precision.HIGHEST
    LO = lax.Precision.DEFAULT
    idx = idx_ref[...]
    d = d_ref[...]
    idxf = idx.astype(jnp.float32)
    kk = lax.broadcasted_iota(jnp.int32, (K, K * K), 0)
    ll = lax.broadcasted_iota(jnp.int32, (K, K * K), 1)
    picki = (ll // K == kk).astype(jnp.float32)
    pickj = (ll % K == kk).astype(jnp.float32)
    diff = jnp.dot(idxf, picki - pickj, precision=HI)
    eq2 = (diff == 0.0).astype(jnp.float32)
    l1 = lax.broadcasted_iota(jnp.int32, (1, K * K), 1)
    ltm = (l1 % K < l1 // K).astype(jnp.float32)
    s0 = lax.broadcasted_iota(jnp.int32, (K * K, K), 0)
    s1 = lax.broadcasted_iota(jnp.int32, (K * K, K), 1)
    sred = (s0 // K == s1).astype(jnp.float32)
    seen = jnp.dot(eq2 * ltm, sred, precision=LO)
    is_new = ((idx != 0) & (seen == 0.0)).astype(jnp.float32)
    r0 = lax.broadcasted_iota(jnp.int32, (K, K), 0)
    r1 = lax.broadcasted_iota(jnp.int32, (K, K), 1)
    tri = (r0 <= r1).astype(jnp.float32)
    counts = jnp.dot(is_new, tri, precision=LO)
    feat = jnp.concatenate([d, counts], axis=-1)
    h = jnp.tanh(jnp.dot(feat, w1_ref[...], precision=HI) + b1_ref[...])
    logit = jnp.sum(h * w2t_ref[...], axis=-1, keepdims=True) + b2_ref[...]
    tempe = jax.nn.sigmoid(logit)
    x = -d * tempe
    x = x - jnp.max(x, axis=-1, keepdims=True)
    e = jnp.exp(x)
    p = e / jnp.sum(e, axis=-1, keepdims=True)
    p2 = jnp.dot(p, pickj, precision=HI)
    comb = jnp.dot(eq2 * p2, sred, precision=HI)
    out_ref[...] = comb


def _tc_combine(idx, d, W1, b1, W2, b2):
    R, K = idx.shape
    return pl.pallas_call(
        _tc_body,
        out_shape=jax.ShapeDtypeStruct((R, K), jnp.float32),
    )(idx, d, W1, b1.reshape(1, -1), W2.reshape(1, -1), b2.reshape(1, 1))


@functools.cache
def _make_sc_scatter(B, S, K, V):
    NC, NS = 2, 16
    NW = NC * NS
    assert B == NW and K % 16 == 0
    CW = 6144
    NCHUNK = V // CW
    assert NCHUNK % 2 == 0
    TW = V - NCHUNK * CW
    mesh = plsc.VectorSubcoreMesh(core_axis_name="c", subcore_axis_name="s")

    def _scatter_halves(buf, idx_v, val_v, base, width, vals_are_zero):
        for s in range(S):
            srow = jnp.full((16,), s, jnp.int32)
            for h in range(K // 16):
                iv = idx_v[s, pl.ds(h * 16, 16)]
                m = (iv >= base) & (iv < base + width)
                loc = jnp.where(m, iv - base, 0)
                if vals_are_zero:
                    vv = jnp.zeros((16,), jnp.float32)
                else:
                    vv = val_v[s, pl.ds(h * 16, 16)]
                plsc.store_scatter(buf, [srow, loc], vv, mask=m)

    @functools.partial(
        pl.kernel,
        mesh=mesh,
        out_type=jax.ShapeDtypeStruct((B, S, V), jnp.float32),
        compiler_params=pltpu.CompilerParams(needs_layout_passes=False),
        scratch_types=[
            pltpu.VMEM((S, CW), jnp.float32),
            pltpu.VMEM((S, CW), jnp.float32),
            pltpu.VMEM((S, TW), jnp.float32),
            pltpu.VMEM((S, K), jnp.int32),
            pltpu.VMEM((S, K), jnp.float32),
            pltpu.SemaphoreType.DMA,
            pltpu.SemaphoreType.DMA,
            pltpu.SemaphoreType.DMA,
        ],
    )
    def sc_scatter(zeros_hbm, idx_hbm, val_hbm, out_hbm,
                   buf_a, buf_b, tailbuf, idx_v, val_v, sem_a, sem_b, sem_p):
        b = lax.axis_index("s") * NC + lax.axis_index("c")
        pre = [
            pltpu.async_copy(zeros_hbm.at[:, pl.ds(0, CW)], buf_a, sem_p),
            pltpu.async_copy(zeros_hbm.at[:, pl.ds(0, CW)], buf_b, sem_p),
            pltpu.async_copy(zeros_hbm.at[:, pl.ds(CW, TW)], tailbuf, sem_p),
            pltpu.async_copy(idx_hbm.at[pl.ds(b * S, S)], idx_v, sem_p),
            pltpu.async_copy(val_hbm.at[pl.ds(b * S, S)], val_v, sem_p),
        ]
        for cp in pre:
            cp.wait()

        def _fire(buf, base, sem):
            return pltpu.async_copy(
                buf, out_hbm.at[b, :, pl.ds(base, CW)], sem)

        _scatter_halves(buf_a, idx_v, val_v, 0, CW, False)
        _fire(buf_a, 0, sem_a)
        _scatter_halves(buf_b, idx_v, val_v, CW, CW, False)
        _fire(buf_b, CW, sem_b)

        @pl.loop(1, NCHUNK // 2)
        def _chunk_pair(i):
            for buf, sem, par in ((buf_a, sem_a, 0), (buf_b, sem_b, 1)):
                base = (2 * i + par) * CW
                pltpu.make_async_copy(
                    buf, out_hbm.at[b, :, pl.ds(base - 2 * CW, CW)], sem
                ).wait()
                _scatter_halves(buf, idx_v, val_v, base - 2 * CW, CW, True)
                _scatter_halves(buf, idx_v, val_v, base, CW, False)
                _fire(buf, base, sem)

        base = NCHUNK * CW
        _scatter_halves(tailbuf, idx_v, val_v, base, TW, False)
        tail_cp = pltpu.async_copy(
            tailbuf, out_hbm.at[b, :, pl.ds(base, TW)], sem_p)
        pltpu.make_async_copy(
            buf_a, out_hbm.at[b, :, pl.ds(0, CW)], sem_a).wait()
        pltpu.make_async_copy(
            buf_b, out_hbm.at[b, :, pl.ds(0, CW)], sem_b).wait()
        tail_cp.wait()

    return sc_scatter


def kernel(tgt_index, knn_dists, nmt_prob, W1, b1, W2, b2):
    B, S, K = knn_dists.shape
    V = nmt_prob.shape[-1]
    R = B * S
    idx = tgt_index.reshape(R, K).astype(jnp.int32)
    d = knn_dists.reshape(R, K).astype(jnp.float32)
    vals = _tc_combine(idx, d, W1, b1, W2, b2)
    CW = 6144
    TW = V - (V // CW) * CW
    zeros_src = jnp.zeros((S, CW + TW), jnp.float32)
    return _make_sc_scatter(B, S, K, V)(zeros_src, idx, vals)


# speedup vs baseline: 1.1106x; 1.0202x over previous
import functools

import jax
import jax.numpy as jnp
from jax import lax
from jax.experimental import pallas as pl
from jax.experimental.pallas import tpu as pltpu
from jax.experimental.pallas import tpu_sc as plsc


def _tc_body(idx_ref, d_ref, w1t_ref, b1_ref, w2t_ref, b2_ref, out_ref):
    R, K = idx_ref.shape
    HI = lax.Precision.HIGHEST
    LO = lax.Precision.DEFAULT
    kk = lax.broadcasted_iota(jnp.int32, (K, K * K), 0)
    ll = lax.broadcasted_iota(jnp.int32, (K, K * K), 1)
    picki = (ll // K == kk).astype(jnp.float32)
    pickj = (ll % K == kk).astype(jnp.float32)
    dmat = picki - pickj
    l1 = lax.broadcasted_iota(jnp.int32, (1, K * K), 1)
    ltm = (l1 % K < l1 // K).astype(jnp.float32)
    s0 = lax.broadcasted_iota(jnp.int32, (K * K, K), 0)
    s1 = lax.broadcasted_iota(jnp.int32, (K * K, K), 1)
    sred = (s0 // K == s1).astype(jnp.float32)
    r0 = lax.broadcasted_iota(jnp.int32, (K, K), 0)
    r1 = lax.broadcasted_iota(jnp.int32, (K, K), 1)
    tri = (r0 <= r1).astype(jnp.float32)
    idx = idx_ref[...]
    d = d_ref[...]
    idxf = idx.astype(jnp.float32)
    diff = jnp.dot(idxf, dmat, precision=HI)
    eq2 = (diff == 0.0).astype(jnp.float32)
    seen = jnp.dot(eq2 * ltm, sred, precision=LO)
    is_new = ((idx != 0) & (seen == 0.0)).astype(jnp.float32)
    counts = jnp.dot(is_new, tri, precision=LO)
    feat = jnp.concatenate([d, counts], axis=-1)
    h = jnp.tanh(
        lax.dot_general(
            feat, w1t_ref[...], (((1,), (1,)), ((), ())), precision=HI)
        + b1_ref[...]
    )
    logit = jnp.sum(h * w2t_ref[...], axis=-1, keepdims=True) + b2_ref[...]
    tempe = jax.nn.sigmoid(logit)
    x = -d * tempe
    x = x - jnp.max(x, axis=-1, keepdims=True)
    e = jnp.exp(x)
    p = e / jnp.sum(e, axis=-1, keepdims=True)
    p2 = jnp.dot(p, pickj, precision=HI)
    comb = jnp.dot(eq2 * p2, sred, precision=HI)
    out_ref[...] = comb


def _tc_combine(idx, d, W1, b1, W2, b2):
    R, K = idx.shape
    return pl.pallas_call(
        _tc_body,
        out_shape=jax.ShapeDtypeStruct((R, K), jnp.float32),
    )(idx, d, W1.T, b1.reshape(1, -1), W2.reshape(1, -1), b2.reshape(1, 1))


@functools.cache
def _make_sc_scatter(B, S, K, V):
    NC, NS = 2, 16
    NW = NC * NS
    assert B == NW and K % 16 == 0
    CW = 6144
    NCHUNK = V // CW
    assert NCHUNK % 2 == 0
    TW = V - NCHUNK * CW
    mesh = plsc.VectorSubcoreMesh(core_axis_name="c", subcore_axis_name="s")

    def _scatter_halves(buf, idx_v, val_v, base, width, vals_are_zero):
        for s in range(S):
            srow = jnp.full((16,), s, jnp.int32)
            for h in range(K // 16):
                iv = idx_v[s, pl.ds(h * 16, 16)]
                m = (iv >= base) & (iv < base + width)
                loc = jnp.where(m, iv - base, 0)
                if vals_are_zero:
                    vv = jnp.zeros((16,), jnp.float32)
                else:
                    vv = val_v[s, pl.ds(h * 16, 16)]
                plsc.store_scatter(buf, [srow, loc], vv, mask=m)

    @functools.partial(
        pl.kernel,
        mesh=mesh,
        out_type=jax.ShapeDtypeStruct((B, S, V), jnp.float32),
        compiler_params=pltpu.CompilerParams(needs_layout_passes=False),
        scratch_types=[
            pltpu.VMEM((S, CW), jnp.float32),
            pltpu.VMEM((S, CW), jnp.float32),
            pltpu.VMEM((S, TW), jnp.float32),
            pltpu.VMEM((S, K), jnp.int32),
            pltpu.VMEM((S, K), jnp.float32),
            pltpu.SemaphoreType.DMA,
            pltpu.SemaphoreType.DMA,
            pltpu.SemaphoreType.DMA,
        ],
    )
    def sc_scatter(zeros_hbm, idx_hbm, val_hbm, out_hbm,
                   buf_a, buf_b, tailbuf, idx_v, val_v, sem_a, sem_b, sem_p):
        b = lax.axis_index("s") * NC + lax.axis_index("c")
        pre = [
            pltpu.async_copy(zeros_hbm.at[:, pl.ds(0, CW)], buf_a, sem_p),
            pltpu.async_copy(zeros_hbm.at[:, pl.ds(0, CW)], buf_b, sem_p),
            pltpu.async_copy(zeros_hbm.at[:, pl.ds(CW, TW)], tailbuf, sem_p),
            pltpu.async_copy(idx_hbm.at[pl.ds(b * S, S)], idx_v, sem_p),
            pltpu.async_copy(val_hbm.at[pl.ds(b * S, S)], val_v, sem_p),
        ]
        for cp in pre:
            cp.wait()

        def _fire(buf, base, sem):
            return pltpu.async_copy(
                buf, out_hbm.at[b, :, pl.ds(base, CW)], sem)

        _scatter_halves(buf_a, idx_v, val_v, 0, CW, False)
        _fire(buf_a, 0, sem_a)
        _scatter_halves(buf_b, idx_v, val_v, CW, CW, False)
        _fire(buf_b, CW, sem_b)

        @pl.loop(1, NCHUNK // 2)
        def _chunk_pair(i):
            for buf, sem, par in ((buf_a, sem_a, 0), (buf_b, sem_b, 1)):
                base = (2 * i + par) * CW
                pltpu.make_async_copy(
                    buf, out_hbm.at[b, :, pl.ds(base - 2 * CW, CW)], sem
                ).wait()
                _scatter_halves(buf, idx_v, val_v, base - 2 * CW, CW, True)
                _scatter_halves(buf, idx_v, val_v, base, CW, False)
                _fire(buf, base, sem)

        base = NCHUNK * CW
        _scatter_halves(tailbuf, idx_v, val_v, base, TW, False)
        tail_cp = pltpu.async_copy(
            tailbuf, out_hbm.at[b, :, pl.ds(base, TW)], sem_p)
        pltpu.make_async_copy(
            buf_a, out_hbm.at[b, :, pl.ds(0, CW)], sem_a).wait()
        pltpu.make_async_copy(
            buf_b, out_hbm.at[b, :, pl.ds(0, CW)], sem_b).wait()
        tail_cp.wait()

    return sc_scatter


def kernel(tgt_index, knn_dists, nmt_prob, W1, b1, W2, b2):
    B, S, K = knn_dists.shape
    V = nmt_prob.shape[-1]
    R = B * S
    idx = tgt_index.reshape(R, K).astype(jnp.int32)
    d = knn_dists.reshape(R, K).astype(jnp.float32)
    vals = _tc_combine(idx, d, W1, b1, W2, b2)
    CW = 6144
    TW = V - (V // CW) * CW
    zeros_src = jnp.zeros((S, CW + TW), jnp.float32)
    return _make_sc_scatter(B, S, K, V)(zeros_src, idx, vals)
